# baseline (speedup 1.0000x reference)
import functools

import jax
import jax.numpy as jnp
from jax import lax
from jax.experimental import pallas as pl
from jax.experimental.pallas import tpu as pltpu
from jax.experimental.pallas import tpu_sc as plsc

B = 4096
L = 200
NC = 2
NS = 16
NW = NC * NS
RPW = B // NW
LANES = 16
NPAIR_SLICES = 7
NSUM_FULL = L // LANES
SUM_TAIL = L - NSUM_FULL * LANES

_mesh = plsc.VectorSubcoreMesh(core_axis_name="c", subcore_axis_name="s")


@functools.partial(
    pl.kernel,
    out_type=jax.ShapeDtypeStruct((NW, LANES), jnp.float32),
    mesh=_mesh,
    scratch_types=[
        pltpu.VMEM((RPW, L), jnp.float32),
        pltpu.VMEM((RPW,), jnp.int32),
        pltpu.VMEM((RPW,), jnp.int32),
        pltpu.VMEM((RPW,), jnp.float32),
        pltpu.VMEM((RPW,), jnp.int32),
        pltpu.VMEM((RPW,), jnp.float32),
        pltpu.VMEM((LANES,), jnp.float32),
    ],
    compiler_params=pltpu.CompilerParams(needs_layout_passes=False),
)
def _sc_partials(x_hbm, il_hbm, un_hbm, y_hbm, out_hbm,
                 xv, ilv, unv, yv, hpv, rsv, stage):
    cid = lax.axis_index("c")
    sid = lax.axis_index("s")
    wid = sid * NC + cid
    rbase = wid * RPW

    pltpu.sync_copy(x_hbm.at[pl.ds(rbase, RPW), :], xv)
    pltpu.sync_copy(il_hbm.at[pl.ds(rbase, RPW)], ilv)
    pltpu.sync_copy(un_hbm.at[pl.ds(rbase, RPW)], unv)
    pltpu.sync_copy(y_hbm.at[pl.ds(rbase, RPW)], yv)

    iota = lax.iota(jnp.int32, LANES)

    pairs_vec = jnp.zeros((LANES,), jnp.int32)
    for g in range(RPW // LANES):
        il = ilv[pl.ds(g * LANES, LANES)]
        un = unv[pl.ds(g * LANES, LANES)]
        hp = jnp.where(un != 0, il >> 1, 0)
        hpv[pl.ds(g * LANES, LANES)] = hp
        pairs_vec = pairs_vec + hp

    def row_body(r, sqacc):
        rvec = jnp.full((LANES,), r, dtype=jnp.int32)
        hpb = plsc.load_gather(hpv, [rvec])
        for j in range(NPAIR_SLICES):
            iidx = iota + (j * LANES)
            pm = iidx < hpb
            f = xv[r, pl.ds(j * LANES, LANES)]
            s = plsc.load_gather(xv, [rvec, iidx + hpb], mask=pm)
            d = f - s
            sqacc = sqacc + jnp.where(pm, d * d, 0.0)
        rs = jnp.zeros((LANES,), jnp.float32)
        for j in range(NSUM_FULL):
            rs = rs + xv[r, pl.ds(j * LANES, LANES)]
        tail = plsc.load_gather(xv, [rvec, iota + NSUM_FULL * LANES],
                                mask=iota < SUM_TAIL)
        rs = rs + jnp.where(iota < SUM_TAIL, tail, 0.0)
        rsum = jnp.sum(rs)
        plsc.store_scatter(rsv, [rvec], jnp.full((LANES,), rsum, jnp.float32),
                           mask=iota == 0)
        return sqacc

    sqacc = lax.fori_loop(0, RPW, row_body, jnp.zeros((LANES,), jnp.float32))

    lossacc = jnp.zeros((LANES,), jnp.float32)
    for g in range(RPW // LANES):
        dv = rsv[pl.ds(g * LANES, LANES)] - yv[pl.ds(g * LANES, LANES)]
        lossacc = lossacc + dv * dv

    sq_s = jnp.sum(sqacc)
    pair_s = jnp.sum(pairs_vec).astype(jnp.float32)
    loss_s = jnp.sum(lossacc)
    v = jnp.where(iota == 0, sq_s,
                  jnp.where(iota == 1, pair_s,
                            jnp.where(iota == 2, loss_s, 0.0)))
    stage[...] = v
    pltpu.sync_copy(stage, out_hbm.at[wid])


def _combine_body(p_ref, o_ref):
    p = p_ref[...]
    sq = jnp.sum(p[:, 0])
    pr = jnp.sum(p[:, 1])
    ls = jnp.sum(p[:, 2])
    total = ls / jnp.float32(B) + sq / jnp.maximum(pr, 1.0)
    o_ref[...] = jnp.full((1, 1), total, dtype=jnp.float32)


def kernel(outputs, y, unrolled, inst_len):
    un = unrolled.astype(jnp.int32)
    part = _sc_partials(outputs, inst_len.astype(jnp.int32), un, y)
    s = part[:, :3].sum(axis=0)
    return s[2] / jnp.float32(B) + s[0] / jnp.maximum(s[1], 1.0)


# speedup vs baseline: 1.0696x; 1.0696x over previous
import functools

import jax
import jax.numpy as jnp
from jax import lax
from jax.experimental import pallas as pl
from jax.experimental.pallas import tpu as pltpu
from jax.experimental.pallas import tpu_sc as plsc

B = 4096
L = 200
NC = 2
NS = 16
NW = NC * NS
RPW = B // NW
LANES = 16
NPAIR_SLICES = 7
NSUM_FULL = L // LANES
SUM_TAIL = L - NSUM_FULL * LANES

_mesh = plsc.VectorSubcoreMesh(core_axis_name="c", subcore_axis_name="s")


@functools.partial(
    pl.kernel,
    out_type=jax.ShapeDtypeStruct((NW, LANES), jnp.float32),
    mesh=_mesh,
    scratch_types=[
        pltpu.VMEM((RPW + 1, L), jnp.float32),
        pltpu.VMEM((RPW,), jnp.int32),
        pltpu.VMEM((RPW,), jnp.int32),
        pltpu.VMEM((RPW,), jnp.float32),
        pltpu.VMEM((RPW + LANES,), jnp.int32),
        pltpu.VMEM((RPW,), jnp.float32),
        pltpu.VMEM((LANES,), jnp.float32),
    ],
    compiler_params=pltpu.CompilerParams(needs_layout_passes=False),
)
def _sc_partials(x_hbm, il_hbm, un_hbm, y_hbm, out_hbm,
                 xv, ilv, unv, yv, hpv, rsv, stage):
    cid = lax.axis_index("c")
    sid = lax.axis_index("s")
    wid = sid * NC + cid
    rbase = wid * RPW

    pltpu.sync_copy(x_hbm.at[pl.ds(rbase, RPW), :], xv.at[pl.ds(0, RPW), :])
    pltpu.sync_copy(il_hbm.at[pl.ds(rbase, RPW)], ilv)
    pltpu.sync_copy(un_hbm.at[pl.ds(rbase, RPW)], unv)
    pltpu.sync_copy(y_hbm.at[pl.ds(rbase, RPW)], yv)

    iota = lax.iota(jnp.int32, LANES)

    pairs_vec = jnp.zeros((LANES,), jnp.int32)
    for g in range(RPW // LANES):
        il = ilv[pl.ds(g * LANES, LANES)]
        un = unv[pl.ds(g * LANES, LANES)]
        hp = jnp.where(un != 0, il >> 1, 0)
        hpv[pl.ds(g * LANES, LANES)] = hp
        pairs_vec = pairs_vec + hp

    def row_body(r, sqacc):
        rvec = jnp.full((LANES,), r, dtype=jnp.int32)
        hs = hpv[pl.ds(r, LANES)][0]
        for j in range(NPAIR_SLICES):
            pmb = jnp.full((LANES,), hs - j * LANES, dtype=jnp.int32)
            pm = iota < pmb
            f = xv[r, pl.ds(j * LANES, LANES)]
            s = xv[r, pl.ds(j * LANES + hs, LANES)]
            d = f - s
            sqacc = sqacc + jnp.where(pm, d * d, 0.0)
        rs = jnp.zeros((LANES,), jnp.float32)
        for j in range(NSUM_FULL):
            rs = rs + xv[r, pl.ds(j * LANES, LANES)]
        tail = xv[r, pl.ds(L - LANES, LANES)]
        rs = rs + jnp.where(iota >= LANES - SUM_TAIL, tail, 0.0)
        rsum = jnp.sum(rs)
        plsc.store_scatter(rsv, [rvec], jnp.full((LANES,), rsum, jnp.float32),
                           mask=iota == 0)
        return sqacc

    sqacc = lax.fori_loop(0, RPW, row_body, jnp.zeros((LANES,), jnp.float32))

    lossacc = jnp.zeros((LANES,), jnp.float32)
    for g in range(RPW // LANES):
        dv = rsv[pl.ds(g * LANES, LANES)] - yv[pl.ds(g * LANES, LANES)]
        lossacc = lossacc + dv * dv

    sq_s = jnp.sum(sqacc)
    pair_s = jnp.sum(pairs_vec).astype(jnp.float32)
    loss_s = jnp.sum(lossacc)
    v = jnp.where(iota == 0, sq_s,
                  jnp.where(iota == 1, pair_s,
                            jnp.where(iota == 2, loss_s, 0.0)))
    stage[...] = v
    pltpu.sync_copy(stage, out_hbm.at[wid])


def _combine_body(p_ref, o_ref):
    p = p_ref[...]
    sq = jnp.sum(p[:, 0])
    pr = jnp.sum(p[:, 1])
    ls = jnp.sum(p[:, 2])
    total = ls / jnp.float32(B) + sq / jnp.maximum(pr, 1.0)
    o_ref[...] = jnp.full((1, 1), total, dtype=jnp.float32)


def kernel(outputs, y, unrolled, inst_len):
    un = unrolled.astype(jnp.int32)
    part = _sc_partials(outputs, inst_len.astype(jnp.int32), un, y)
    total = pl.pallas_call(
        _combine_body,
        out_shape=jax.ShapeDtypeStruct((1, 1), jnp.float32),
    )(part)
    return total[0, 0]
